# baseline (speedup 1.0000x reference)
import jax
import jax.numpy as jnp
from jax.experimental import pallas as pl
from jax.experimental.pallas import tpu as pltpu

ROWS, D = 8192, 768
CHUNK = 512
NCHUNK = ROWS // CHUNK
NBUF = 16


def _body(w_ref, o_ref, buf, sem_in, sem_out):
    ins = [
        pltpu.make_async_copy(
            w_ref.at[pl.ds(i * CHUNK, CHUNK)], buf.at[i % NBUF], sem_in.at[i % NBUF]
        )
        for i in range(NCHUNK)
    ]
    outs = [
        pltpu.make_async_copy(
            buf.at[i % NBUF], o_ref.at[pl.ds(i * CHUNK, CHUNK)], sem_out.at[i % NBUF]
        )
        for i in range(NCHUNK)
    ]
    for i in range(NBUF):
        ins[i].start()
    for i in range(NCHUNK):
        if 0 < i and i - 1 + NBUF < NCHUNK:
            outs[i - 1].wait()
            ins[i - 1 + NBUF].start()
        ins[i].wait()
        outs[i].start()
    for i in range(max(0, NCHUNK - NBUF), NCHUNK):
        outs[i].wait()


def kernel(x, W):
    del x
    return pl.pallas_call(
        _body,
        in_specs=[pl.BlockSpec(memory_space=pl.ANY)],
        out_specs=pl.BlockSpec(memory_space=pl.ANY),
        out_shape=jax.ShapeDtypeStruct((ROWS, D), jnp.float32),
        scratch_shapes=[
            pltpu.VMEM((NBUF, CHUNK, D), jnp.float32),
            pltpu.SemaphoreType.DMA((NBUF,)),
            pltpu.SemaphoreType.DMA((NBUF,)),
        ],
    )(W)


# speedup vs baseline: 1.0340x; 1.0340x over previous
import jax
import jax.numpy as jnp
from jax.experimental import pallas as pl
from jax.experimental.pallas import tpu as pltpu

ROWS, D = 8192, 768
CHUNK = 4096
NCHUNK = ROWS // CHUNK
NBUF = 2


def _body(w_ref, o_ref, buf, sem_in, sem_out):
    ins = [
        pltpu.make_async_copy(
            w_ref.at[pl.ds(i * CHUNK, CHUNK)], buf.at[i % NBUF], sem_in.at[i % NBUF]
        )
        for i in range(NCHUNK)
    ]
    outs = [
        pltpu.make_async_copy(
            buf.at[i % NBUF], o_ref.at[pl.ds(i * CHUNK, CHUNK)], sem_out.at[i % NBUF]
        )
        for i in range(NCHUNK)
    ]
    for i in range(NBUF):
        ins[i].start()
    for i in range(NCHUNK):
        if 0 < i and i - 1 + NBUF < NCHUNK:
            outs[i - 1].wait()
            ins[i - 1 + NBUF].start()
        ins[i].wait()
        outs[i].start()
    for i in range(max(0, NCHUNK - NBUF), NCHUNK):
        outs[i].wait()


def kernel(x, W):
    del x
    return pl.pallas_call(
        _body,
        in_specs=[pl.BlockSpec(memory_space=pl.ANY)],
        out_specs=pl.BlockSpec(memory_space=pl.ANY),
        out_shape=jax.ShapeDtypeStruct((ROWS, D), jnp.float32),
        scratch_shapes=[
            pltpu.VMEM((NBUF, CHUNK, D), jnp.float32),
            pltpu.SemaphoreType.DMA((NBUF,)),
            pltpu.SemaphoreType.DMA((NBUF,)),
        ],
    )(W)
